# baseline (speedup 1.0000x reference)
import dataclasses
import functools

import jax
import jax.numpy as jnp
from jax import lax
from jax.experimental import pallas as pl
from jax.experimental.pallas import tpu as pltpu
from jax.experimental.pallas import tpu_sc as plsc

N = 16384
B = 64
NNZ = 268435

NC = 2
NS = 16
NW = NC * NS
L = 16

C = 128
NBUF = 4
PER_TILE = -(-NNZ // (NW * C * NBUF)) * C * NBUF
NNZ_PAD = PER_TILE * NW
CHUNKS = PER_TILE // C
OUTER = CHUNKS // NBUF
STRIPE = N // NS


def _sc_compiler_params():
    cp = pltpu.CompilerParams()
    if "needs_layout_passes" in pltpu.CompilerParams.__dataclass_fields__:
        cp = dataclasses.replace(cp, needs_layout_passes=False)
    cp = dataclasses.replace(cp, use_tc_tiling_on_sc=False)
    return cp


def _sc_spmm(x_t, rows3, cols3, vals3):
    mesh = plsc.VectorSubcoreMesh(core_axis_name="c", subcore_axis_name="s")

    @functools.partial(
        pl.kernel,
        compiler_params=_sc_compiler_params(),
        out_type=jax.ShapeDtypeStruct((NC, N, B), jnp.float32),
        mesh=mesh,
        scratch_types=[
            pltpu.VMEM((CHUNKS, C), jnp.int32),
            pltpu.VMEM((CHUNKS, C), jnp.int32),
            pltpu.VMEM((CHUNKS, C), jnp.float32),
            pltpu.VMEM((NBUF, C, B), jnp.float32),
            pltpu.VMEM_SHARED((N, B), jnp.float32),
        ]
        + [pltpu.SemaphoreType.DMA] * (2 * NBUF + 1),
    )
    def k(xt_hbm, rows_hbm, cols_hbm, vals_hbm, out_hbm,
          rows_v, cols_v, vals_v, gbufs, acc, *sems):
        gsem = sems[:NBUF]
        ssem = sems[NBUF:2 * NBUF]
        sem = sems[2 * NBUF]
        c = lax.axis_index("c")
        s = lax.axis_index("s")
        wid = c * NS + s

        zero = jnp.zeros((L,), jnp.float32)

        @pl.loop(0, C)
        def _(i):
            for q in range(B // L):
                gbufs[0, i, pl.ds(q * L, L)] = zero

        @pl.loop(0, STRIPE // C)
        def _(i):
            pltpu.sync_copy(gbufs.at[0], acc.at[pl.ds(s * STRIPE + i * C, C)])

        pltpu.sync_copy(rows_hbm.at[wid], rows_v)
        pltpu.sync_copy(cols_hbm.at[wid], cols_v)
        pltpu.sync_copy(vals_hbm.at[wid], vals_v)

        plsc.subcore_barrier()

        def start_gather(j, b):
            pltpu.async_copy(xt_hbm.at[cols_v.at[j]], gbufs.at[b], gsem[b])

        def wait_gather(j, b):
            pltpu.make_async_copy(
                xt_hbm.at[cols_v.at[j]], gbufs.at[b], gsem[b]).wait()

        def start_scatter(j, b):
            pltpu.async_copy(gbufs.at[b], acc.at[rows_v.at[j]], ssem[b],
                             add=True)

        def wait_scatter(j, b):
            pltpu.make_async_copy(
                gbufs.at[b], acc.at[rows_v.at[j]], ssem[b]).wait()

        start_gather(0, 0)
        start_gather(1, 1)

        @pl.loop(0, OUTER)
        def _(o):
            for b in range(NBUF):
                j = o * NBUF + b
                b2 = (b + 2) % NBUF
                j2 = j + 2
                if b < 2:
                    @pl.when(o > 0)
                    def _():
                        wait_scatter(j2 - NBUF, b2)
                    start_gather(j2, b2)
                else:
                    wait_scatter(j2 - NBUF, b2)

                    @pl.when(j2 < CHUNKS)
                    def _():
                        start_gather(j2, b2)

                wait_gather(j, b)

                @pl.loop(0, C)
                def _(i):
                    v = plsc.load_gather(
                        vals_v,
                        [jnp.full((L,), j, jnp.int32),
                         jnp.full((L,), i, jnp.int32)],
                    )
                    for q in range(B // L):
                        sl = (b, i, pl.ds(q * L, L))
                        gbufs[sl] = gbufs[sl] * v

                start_scatter(j, b)

        wait_scatter(CHUNKS - 2, (CHUNKS - 2) % NBUF)
        wait_scatter(CHUNKS - 1, (CHUNKS - 1) % NBUF)

        plsc.subcore_barrier()

        pltpu.sync_copy(acc.at[pl.ds(s * STRIPE, STRIPE)],
                        out_hbm.at[c].at[pl.ds(s * STRIPE, STRIPE)])

    return k(x_t, rows3, cols3, vals3)


_TN = 512


def _tc_combine(partials):
    def body(p_ref, o_ref):
        o_ref[...] = (p_ref[0] + p_ref[1]).T

    return pl.pallas_call(
        body,
        grid=(N // _TN,),
        in_specs=[pl.BlockSpec((NC, _TN, B), lambda i: (0, i, 0))],
        out_specs=pl.BlockSpec((B, _TN), lambda i: (0, i)),
        out_shape=jax.ShapeDtypeStruct((B, N), jnp.float32),
    )(partials)


@jax.jit
def kernel(x_affine, rows, cols, vals):
    pad = NNZ_PAD - NNZ
    rows3 = jnp.pad(rows, (0, pad)).reshape(NW, CHUNKS, C)
    cols3 = jnp.pad(cols, (0, pad)).reshape(NW, CHUNKS, C)
    vals3 = jnp.pad(vals, (0, pad)).reshape(NW, CHUNKS, C)
    x_t = x_affine.T
    partials = _sc_spmm(x_t, rows3, cols3, vals3)
    return _tc_combine(partials)


# speedup vs baseline: 1.8059x; 1.8059x over previous
import dataclasses
import functools

import jax
import jax.numpy as jnp
from jax import lax
from jax.experimental import pallas as pl
from jax.experimental.pallas import tpu as pltpu
from jax.experimental.pallas import tpu_sc as plsc

N = 16384
B = 64
NNZ = 268435

NC = 2
NS = 16
NW = NC * NS
L = 16

C = 128
NBUF = 2
PER_TILE = -(-NNZ // (NW * C * NBUF)) * C * NBUF
NNZ_PAD = PER_TILE * NW
CHUNKS = PER_TILE // C
OUTER = CHUNKS // NBUF
STRIPE = N // NS


def _sc_compiler_params():
    cp = pltpu.CompilerParams()
    if "needs_layout_passes" in pltpu.CompilerParams.__dataclass_fields__:
        cp = dataclasses.replace(cp, needs_layout_passes=False)
    cp = dataclasses.replace(cp, use_tc_tiling_on_sc=False)
    return cp


def _sc_spmm(x_t, rows3, cols3, vals3):
    mesh = plsc.VectorSubcoreMesh(core_axis_name="c", subcore_axis_name="s")

    @functools.partial(
        pl.kernel,
        compiler_params=_sc_compiler_params(),
        out_type=jax.ShapeDtypeStruct((NC, N, B), jnp.float32),
        mesh=mesh,
        scratch_types=[
            pltpu.VMEM((CHUNKS, C), jnp.int32),
            pltpu.VMEM((CHUNKS, C), jnp.int32),
            pltpu.VMEM((CHUNKS, C), jnp.float32),
            pltpu.VMEM((NBUF, C, B), jnp.float32),
            pltpu.VMEM_SHARED((N, B), jnp.float32),
        ]
        + [pltpu.SemaphoreType.DMA] * (2 * NBUF + 1),
    )
    def k(xt_hbm, rows_hbm, cols_hbm, vals_hbm, out_hbm,
          rows_v, cols_v, vals_v, gbufs, acc, *sems):
        gsem = sems[:NBUF]
        ssem = sems[NBUF:2 * NBUF]
        sem = sems[2 * NBUF]
        c = lax.axis_index("c")
        s = lax.axis_index("s")
        wid = c * NS + s

        zero = jnp.zeros((L,), jnp.float32)

        @pl.loop(0, C)
        def _(i):
            for q in range(B // L):
                gbufs[0, i, pl.ds(q * L, L)] = zero

        @pl.loop(0, STRIPE // C)
        def _(i):
            pltpu.sync_copy(gbufs.at[0], acc.at[pl.ds(s * STRIPE + i * C, C)])

        pltpu.sync_copy(rows_hbm.at[wid], rows_v)
        pltpu.sync_copy(cols_hbm.at[wid], cols_v)
        pltpu.sync_copy(vals_hbm.at[wid], vals_v)

        plsc.subcore_barrier()

        def start_gather(j, b):
            pltpu.async_copy(xt_hbm.at[cols_v.at[j]], gbufs.at[b], gsem[b])

        def wait_gather(j, b):
            pltpu.make_async_copy(
                xt_hbm.at[cols_v.at[j]], gbufs.at[b], gsem[b]).wait()

        def start_scatter(j, b):
            pltpu.async_copy(gbufs.at[b], acc.at[rows_v.at[j]], ssem[b],
                             add=True)

        def wait_scatter(j, b):
            pltpu.make_async_copy(
                gbufs.at[b], acc.at[rows_v.at[j]], ssem[b]).wait()

        for b in range(NBUF):
            start_gather(b, b)

        @pl.loop(0, OUTER)
        def _(o):
            for b in range(NBUF):
                j = o * NBUF + b
                wait_gather(j, b)

                @pl.loop(0, C)
                def _(i):
                    v = plsc.load_gather(
                        vals_v,
                        [jnp.full((L,), j, jnp.int32),
                         jnp.full((L,), i, jnp.int32)],
                    )
                    for q in range(B // L):
                        sl = (b, i, pl.ds(q * L, L))
                        gbufs[sl] = gbufs[sl] * v

                pltpu.sync_copy(gbufs.at[b], acc.at[rows_v.at[j]], add=True)

                @pl.when(j + NBUF < CHUNKS)
                def _():
                    start_gather(j + NBUF, b)

        plsc.subcore_barrier()

        pltpu.sync_copy(acc.at[pl.ds(s * STRIPE, STRIPE)],
                        out_hbm.at[c].at[pl.ds(s * STRIPE, STRIPE)])

    return k(x_t, rows3, cols3, vals3)


_TN = 512


def _tc_combine(partials):
    def body(p_ref, o_ref):
        o_ref[...] = (p_ref[0] + p_ref[1]).T

    return pl.pallas_call(
        body,
        grid=(N // _TN,),
        in_specs=[pl.BlockSpec((NC, _TN, B), lambda i: (0, i, 0))],
        out_specs=pl.BlockSpec((B, _TN), lambda i: (0, i)),
        out_shape=jax.ShapeDtypeStruct((B, N), jnp.float32),
    )(partials)


@jax.jit
def kernel(x_affine, rows, cols, vals):
    pad = NNZ_PAD - NNZ
    rows3 = jnp.pad(rows, (0, pad)).reshape(NW, CHUNKS, C)
    cols3 = jnp.pad(cols, (0, pad)).reshape(NW, CHUNKS, C)
    vals3 = jnp.pad(vals, (0, pad)).reshape(NW, CHUNKS, C)
    x_t = x_affine.T
    partials = _sc_spmm(x_t, rows3, cols3, vals3)
    return _tc_combine(partials)
